# Initial kernel scaffold; baseline (speedup 1.0000x reference)
import functools

import jax
import jax.numpy as jnp
from jax import lax
from jax.experimental import pallas as pl
from jax.experimental.pallas import tpu as pltpu
from jax.experimental.pallas import tpu_sc as plsc

_N = 10000
_E = 320000
_DH = 256
_G = 64
_EPS = 1e-5
_NR = 10240
_TRASH = _N
_BM = 1000
_GRID = _N // _BM


def _sc_scatter(table, src3, dst3, zeros128, n_chunks):
    mesh = plsc.VectorSubcoreMesh(core_axis_name="c", subcore_axis_name="s")

    def body(table_h, src_h, dst_h, zeros_h, out_h, src_v, dst_v, buf, acc, sem):
        c = lax.axis_index("c")
        s = lax.axis_index("s")
        pltpu.sync_copy(zeros_h, buf)
        for j in range(_NR // 16 // 128):
            pltpu.sync_copy(buf, acc.at[pl.ds(s * (_NR // 16) + j * 128, 128)])
        pltpu.sync_copy(src_h.at[c, s], src_v)
        pltpu.sync_copy(dst_h.at[c, s], dst_v)
        plsc.subcore_barrier()

        def step(i, carry):
            pltpu.async_copy(table_h.at[src_v.at[i]], buf, sem).wait()
            pltpu.sync_copy(buf, acc.at[dst_v.at[i]], add=True)
            return carry

        lax.fori_loop(0, n_chunks, step, 0)
        plsc.subcore_barrier()
        base = s * (_N // 16)
        for j in range(5):
            r0 = base + j * 125
            pltpu.sync_copy(acc.at[pl.ds(r0, 125)], buf.at[pl.ds(0, 125)])
            pltpu.sync_copy(buf.at[pl.ds(0, 125)], out_h.at[c, pl.ds(r0, 125)])

    k = pl.kernel(
        body,
        out_type=jax.ShapeDtypeStruct((2, _N, 128), jnp.float32),
        mesh=mesh,
        scratch_types=[
            pltpu.VMEM((n_chunks, 128), jnp.int32),
            pltpu.VMEM((n_chunks, 128), jnp.int32),
            pltpu.VMEM((128, 128), jnp.float32),
            pltpu.VMEM_SHARED((_NR, 128), jnp.float32),
            pltpu.SemaphoreType.DMA,
        ],
    )
    return k(table, src3, dst3, zeros128)


def _tc_layer(h, agg, W1, b1, W2, b2, first):
    din = 128 if first else _DH

    def body(h_ref, a_ref, W1_ref, b1_ref, W2_ref, b2_ref, z2_ref, st_ref):
        i = pl.program_id(0)
        if first:
            z = h_ref[...] + a_ref[0] + a_ref[1]
        else:
            z = jnp.concatenate([h_ref[0] + a_ref[0], h_ref[1] + a_ref[1]], axis=1)
        z1 = jnp.maximum(
            jnp.dot(z, W1_ref[...], preferred_element_type=jnp.float32) + b1_ref[...], 0.0)
        z2 = jnp.dot(z1, W2_ref[...], preferred_element_type=jnp.float32) + b2_ref[...]
        z2_ref[...] = z2
        sblk = jnp.concatenate(
            [jnp.sum(z2, axis=0, keepdims=True),
             jnp.sum(z2 * z2, axis=0, keepdims=True),
             jnp.zeros((6, _DH), jnp.float32)], axis=0)

        @pl.when(i == 0)
        def _():
            st_ref[...] = sblk

        @pl.when(i != 0)
        def _():
            st_ref[...] = st_ref[...] + sblk

    if first:
        h_spec = pl.BlockSpec((_BM, din), lambda i: (i, 0))
    else:
        h_spec = pl.BlockSpec((2, _BM, 128), lambda i: (0, i, 0))
    return pl.pallas_call(
        body,
        grid=(_GRID,),
        in_specs=[
            h_spec,
            pl.BlockSpec((2, _BM, 128), lambda i: (0, i, 0)),
            pl.BlockSpec((din, _DH), lambda i: (0, 0)),
            pl.BlockSpec((1, _DH), lambda i: (0, 0)),
            pl.BlockSpec((_DH, _DH), lambda i: (0, 0)),
            pl.BlockSpec((1, _DH), lambda i: (0, 0)),
        ],
        out_specs=[
            pl.BlockSpec((_BM, _DH), lambda i: (i, 0)),
            pl.BlockSpec((8, _DH), lambda i: (0, 0)),
        ],
        out_shape=[
            jax.ShapeDtypeStruct((_N, _DH), jnp.float32),
            jax.ShapeDtypeStruct((8, _DH), jnp.float32),
        ],
    )(h, agg, W1, b1.reshape(1, _DH), W2, b2.reshape(1, _DH))


def _tc_bn(z2, st, gamma, beta):
    def body(z_ref, st_ref, g_ref, be_ref, o_ref):
        mean = st_ref[0] * (1.0 / _N)
        var = st_ref[1] * (1.0 / _N) - mean * mean
        inv = g_ref[0] * lax.rsqrt(var + _EPS)
        sh = be_ref[0] - mean * inv
        y = jnp.maximum(z_ref[...] * inv + sh, 0.0)
        o_ref[0] = y[:, :128]
        o_ref[1] = y[:, 128:]

    return pl.pallas_call(
        body,
        grid=(_GRID,),
        in_specs=[
            pl.BlockSpec((_BM, _DH), lambda i: (i, 0)),
            pl.BlockSpec((8, _DH), lambda i: (0, 0)),
            pl.BlockSpec((1, _DH), lambda i: (0, 0)),
            pl.BlockSpec((1, _DH), lambda i: (0, 0)),
        ],
        out_specs=pl.BlockSpec((2, _BM, 128), lambda i: (0, i, 0)),
        out_shape=jax.ShapeDtypeStruct((2, _N, 128), jnp.float32),
    )(z2, st, gamma.reshape(1, _DH), beta.reshape(1, _DH))


def _tc_pool(h, batch3, fc_W, fc_b):
    def body(h_ref, b_ref, W_ref, bb_ref, o_ref, sums, cnts):
        i = pl.program_id(0)
        hcat = jnp.concatenate([h_ref[0], h_ref[1]], axis=1)
        bcol = b_ref[0, 0, :].reshape(_BM, 1)
        gid = lax.broadcasted_iota(jnp.int32, (_BM, _G), 1)
        P = (bcol == gid).astype(jnp.float32)
        ps = lax.dot_general(P, hcat, (((0,), (0,)), ((), ())),
                             preferred_element_type=jnp.float32)
        pc = lax.dot_general(P, jnp.ones((_BM, 128), jnp.float32),
                             (((0,), (0,)), ((), ())),
                             preferred_element_type=jnp.float32)

        @pl.when(i == 0)
        def _():
            sums[...] = ps
            cnts[...] = pc

        @pl.when(i != 0)
        def _():
            sums[...] = sums[...] + ps
            cnts[...] = cnts[...] + pc

        @pl.when(i == _GRID - 1)
        def _():
            cnt = jnp.maximum(cnts[:, 0:1], 1.0)
            pooled = sums[...] / cnt
            o_ref[...] = jnp.dot(pooled, W_ref[...],
                                 preferred_element_type=jnp.float32) + bb_ref[...]

    return pl.pallas_call(
        body,
        grid=(_GRID,),
        in_specs=[
            pl.BlockSpec((2, _BM, 128), lambda i: (0, i, 0)),
            pl.BlockSpec((1, 1, _BM), lambda i: (i, 0, 0)),
            pl.BlockSpec((_DH, 128), lambda i: (0, 0)),
            pl.BlockSpec((1, 128), lambda i: (0, 0)),
        ],
        out_specs=pl.BlockSpec((_G, 128), lambda i: (0, 0)),
        out_shape=jax.ShapeDtypeStruct((_G, 128), jnp.float32),
        scratch_shapes=[
            pltpu.VMEM((_G, _DH), jnp.float32),
            pltpu.VMEM((_G, 128), jnp.float32),
        ],
    )(h, batch3, fc_W, fc_b.reshape(1, 128))


def kernel(x, edge_index, batch,
           l0_W1, l0_b1, l0_W2, l0_b2, l0_gamma, l0_beta,
           l1_W1, l1_b1, l1_W2, l1_b2, l1_gamma, l1_beta,
           l2_W1, l2_b1, l2_W2, l2_b2, l2_gamma, l2_beta,
           fc_W, fc_b):
    src = edge_index[0]
    dst = edge_index[1]

    srcA = jnp.pad(src.reshape(2, 16, _E // 32), ((0, 0), (0, 0), (0, 112)),
                   constant_values=0).reshape(2, 16, 79, 128)
    dstA = jnp.pad(dst.reshape(2, 16, _E // 32), ((0, 0), (0, 0), (0, 112)),
                   constant_values=_TRASH).reshape(2, 16, 79, 128)
    sb = jnp.pad(src.reshape(16, _E // 16), ((0, 0), (0, 96)),
                 constant_values=0).reshape(16, 157, 128)
    db = jnp.pad(dst.reshape(16, _E // 16), ((0, 0), (0, 96)),
                 constant_values=_TRASH).reshape(16, 157, 128)
    srcB = jnp.stack([sb, sb + _N])
    dstB = jnp.stack([db, db])
    zeros128 = jnp.zeros((128, 128), jnp.float32)

    agg0 = _sc_scatter(x, srcA, dstA, zeros128, 79)
    z2, st = _tc_layer(x, agg0, l0_W1, l0_b1, l0_W2, l0_b2, first=True)
    h = _tc_bn(z2, st, l0_gamma, l0_beta)

    agg1 = _sc_scatter(h.reshape(2 * _N, 128), srcB, dstB, zeros128, 157)
    z2, st = _tc_layer(h, agg1, l1_W1, l1_b1, l1_W2, l1_b2, first=False)
    h = _tc_bn(z2, st, l1_gamma, l1_beta)

    agg2 = _sc_scatter(h.reshape(2 * _N, 128), srcB, dstB, zeros128, 157)
    z2, st = _tc_layer(h, agg2, l2_W1, l2_b1, l2_W2, l2_b2, first=False)
    h = _tc_bn(z2, st, l2_gamma, l2_beta)

    return _tc_pool(h, batch.reshape(_GRID, 1, _BM), fc_W, fc_b)


# speedup vs baseline: 3.1912x; 3.1912x over previous
import functools

import jax
import jax.numpy as jnp
from jax import lax
from jax.experimental import pallas as pl
from jax.experimental.pallas import tpu as pltpu
from jax.experimental.pallas import tpu_sc as plsc

_N = 10000
_E = 320000
_DH = 256
_G = 64
_EPS = 1e-5
_NR = 10240
_TRASH = _N
_BM = 1000
_GRID = _N // _BM
_NB = 16


def _sc_scatter(table, src3, dst3, zeros128, n_chunks):
    mesh = plsc.VectorSubcoreMesh(core_axis_name="c", subcore_axis_name="s")

    def body(table_h, src_h, dst_h, zeros_h, out_h, src_v, dst_v, buf, acc, sem):
        c = lax.axis_index("c")
        s = lax.axis_index("s")
        pltpu.sync_copy(zeros_h, buf)
        for j in range(_NR // 16 // 128):
            pltpu.sync_copy(buf, acc.at[pl.ds(s * (_NR // 16) + j * 128, 128)])
        plsc.subcore_barrier()

        def step(i, carry):
            pltpu.async_copy(table_h.at[src_v.at[i]], buf, sem).wait()
            pltpu.sync_copy(buf, acc.at[dst_v.at[i]], add=True)
            return carry

        for b in range(n_chunks // _NB):
            pltpu.sync_copy(src_h.at[c, s, pl.ds(b * _NB, _NB)], src_v)
            pltpu.sync_copy(dst_h.at[c, s, pl.ds(b * _NB, _NB)], dst_v)
            lax.fori_loop(0, _NB, step, 0)
        plsc.subcore_barrier()
        base = s * (_NR // 16)
        for j in range(_NR // 16 // 128):
            r0 = base + j * 128
            pltpu.sync_copy(acc.at[pl.ds(r0, 128)], buf)
            pltpu.sync_copy(buf, out_h.at[c, pl.ds(r0, 128)])

    k = pl.kernel(
        body,
        out_type=jax.ShapeDtypeStruct((2, _NR, 128), jnp.float32),
        mesh=mesh,
        scratch_types=[
            pltpu.VMEM((_NB, 128), jnp.int32),
            pltpu.VMEM((_NB, 128), jnp.int32),
            pltpu.VMEM((128, 128), jnp.float32),
            pltpu.VMEM_SHARED((_NR, 128), jnp.float32),
            pltpu.SemaphoreType.DMA,
        ],
    )
    return k(table, src3, dst3, zeros128)


def _tc_layer(h, agg, W1, b1, W2, b2, first):
    din = 128 if first else _DH

    def body(h_ref, a_ref, W1_ref, b1_ref, W2_ref, b2_ref, z2_ref, st_ref):
        i = pl.program_id(0)
        if first:
            z = h_ref[...] + a_ref[0] + a_ref[1]
        else:
            z = jnp.concatenate([h_ref[0] + a_ref[0], h_ref[1] + a_ref[1]], axis=1)
        z1 = jnp.maximum(
            jnp.dot(z, W1_ref[...], preferred_element_type=jnp.float32) + b1_ref[...], 0.0)
        z2 = jnp.dot(z1, W2_ref[...], preferred_element_type=jnp.float32) + b2_ref[...]
        z2_ref[...] = z2
        sblk = jnp.concatenate(
            [jnp.sum(z2, axis=0, keepdims=True),
             jnp.sum(z2 * z2, axis=0, keepdims=True),
             jnp.zeros((6, _DH), jnp.float32)], axis=0)

        @pl.when(i == 0)
        def _():
            st_ref[...] = sblk

        @pl.when(i != 0)
        def _():
            st_ref[...] = st_ref[...] + sblk

    if first:
        h_spec = pl.BlockSpec((_BM, din), lambda i: (i, 0))
    else:
        h_spec = pl.BlockSpec((2, _BM, 128), lambda i: (0, i, 0))
    return pl.pallas_call(
        body,
        grid=(_GRID,),
        in_specs=[
            h_spec,
            pl.BlockSpec((2, _BM, 128), lambda i: (0, i, 0)),
            pl.BlockSpec((din, _DH), lambda i: (0, 0)),
            pl.BlockSpec((1, _DH), lambda i: (0, 0)),
            pl.BlockSpec((_DH, _DH), lambda i: (0, 0)),
            pl.BlockSpec((1, _DH), lambda i: (0, 0)),
        ],
        out_specs=[
            pl.BlockSpec((_BM, _DH), lambda i: (i, 0)),
            pl.BlockSpec((8, _DH), lambda i: (0, 0)),
        ],
        out_shape=[
            jax.ShapeDtypeStruct((_N, _DH), jnp.float32),
            jax.ShapeDtypeStruct((8, _DH), jnp.float32),
        ],
    )(h, agg, W1, b1.reshape(1, _DH), W2, b2.reshape(1, _DH))


def _tc_bn(z2, st, gamma, beta):
    def body(z_ref, st_ref, g_ref, be_ref, o_ref):
        mean = st_ref[0] * (1.0 / _N)
        var = st_ref[1] * (1.0 / _N) - mean * mean
        inv = g_ref[0] * lax.rsqrt(var + _EPS)
        sh = be_ref[0] - mean * inv
        y = jnp.maximum(z_ref[...] * inv + sh, 0.0)
        o_ref[0] = y[:, :128]
        o_ref[1] = y[:, 128:]

    return pl.pallas_call(
        body,
        grid=(_GRID,),
        in_specs=[
            pl.BlockSpec((_BM, _DH), lambda i: (i, 0)),
            pl.BlockSpec((8, _DH), lambda i: (0, 0)),
            pl.BlockSpec((1, _DH), lambda i: (0, 0)),
            pl.BlockSpec((1, _DH), lambda i: (0, 0)),
        ],
        out_specs=pl.BlockSpec((2, _BM, 128), lambda i: (0, i, 0)),
        out_shape=jax.ShapeDtypeStruct((2, _N, 128), jnp.float32),
    )(z2, st, gamma.reshape(1, _DH), beta.reshape(1, _DH))


def _tc_pool(h, batch3, fc_W, fc_b):
    def body(h_ref, b_ref, W_ref, bb_ref, o_ref, sums, cnts):
        i = pl.program_id(0)
        hcat = jnp.concatenate([h_ref[0], h_ref[1]], axis=1)
        bcol = b_ref[0, 0, :].reshape(_BM, 1)
        gid = lax.broadcasted_iota(jnp.int32, (_BM, _G), 1)
        P = (bcol == gid).astype(jnp.float32)
        ps = lax.dot_general(P, hcat, (((0,), (0,)), ((), ())),
                             preferred_element_type=jnp.float32)
        pc = lax.dot_general(P, jnp.ones((_BM, 128), jnp.float32),
                             (((0,), (0,)), ((), ())),
                             preferred_element_type=jnp.float32)

        @pl.when(i == 0)
        def _():
            sums[...] = ps
            cnts[...] = pc

        @pl.when(i != 0)
        def _():
            sums[...] = sums[...] + ps
            cnts[...] = cnts[...] + pc

        @pl.when(i == _GRID - 1)
        def _():
            cnt = jnp.maximum(cnts[:, 0:1], 1.0)
            pooled = sums[...] / cnt
            o_ref[...] = jnp.dot(pooled, W_ref[...],
                                 preferred_element_type=jnp.float32) + bb_ref[...]

    return pl.pallas_call(
        body,
        grid=(_GRID,),
        in_specs=[
            pl.BlockSpec((2, _BM, 128), lambda i: (0, i, 0)),
            pl.BlockSpec((1, 1, _BM), lambda i: (i, 0, 0)),
            pl.BlockSpec((_DH, 128), lambda i: (0, 0)),
            pl.BlockSpec((1, 128), lambda i: (0, 0)),
        ],
        out_specs=pl.BlockSpec((_G, 128), lambda i: (0, 0)),
        out_shape=jax.ShapeDtypeStruct((_G, 128), jnp.float32),
        scratch_shapes=[
            pltpu.VMEM((_G, _DH), jnp.float32),
            pltpu.VMEM((_G, 128), jnp.float32),
        ],
    )(h, batch3, fc_W, fc_b.reshape(1, 128))


def kernel(x, edge_index, batch,
           l0_W1, l0_b1, l0_W2, l0_b2, l0_gamma, l0_beta,
           l1_W1, l1_b1, l1_W2, l1_b2, l1_gamma, l1_beta,
           l2_W1, l2_b1, l2_W2, l2_b2, l2_gamma, l2_beta,
           fc_W, fc_b):
    order = jnp.argsort(edge_index[1])
    src = edge_index[0][order]
    dst = edge_index[1][order]

    def strided(v, pad_val, n_chunks_total):
        m = n_chunks_total * 128
        vp = jnp.pad(v, (0, m - v.shape[0]), constant_values=pad_val)
        return vp.reshape(128, n_chunks_total).T

    srcA = jnp.stack([strided(src[:_E // 2], 0, 1280),
                      strided(src[_E // 2:], 0, 1280)]).reshape(2, 16, 80, 128)
    dstA = jnp.stack([strided(dst[:_E // 2], _TRASH, 1280),
                      strided(dst[_E // 2:], _TRASH, 1280)]).reshape(2, 16, 80, 128)
    sb = strided(src, 0, 2560).reshape(16, 160, 128)
    db = strided(dst, _TRASH, 2560).reshape(16, 160, 128)
    srcB = jnp.stack([sb, sb + _N])
    dstB = jnp.stack([db, db])
    zeros128 = jnp.zeros((128, 128), jnp.float32)

    agg0 = _sc_scatter(x, srcA, dstA, zeros128, 80)
    z2, st = _tc_layer(x, agg0, l0_W1, l0_b1, l0_W2, l0_b2, first=True)
    h = _tc_bn(z2, st, l0_gamma, l0_beta)

    agg1 = _sc_scatter(h.reshape(2 * _N, 128), srcB, dstB, zeros128, 160)
    z2, st = _tc_layer(h, agg1, l1_W1, l1_b1, l1_W2, l1_b2, first=False)
    h = _tc_bn(z2, st, l1_gamma, l1_beta)

    agg2 = _sc_scatter(h.reshape(2 * _N, 128), srcB, dstB, zeros128, 160)
    z2, st = _tc_layer(h, agg2, l2_W1, l2_b1, l2_W2, l2_b2, first=False)
    h = _tc_bn(z2, st, l2_gamma, l2_beta)

    return _tc_pool(h, batch.reshape(_GRID, 1, _BM), fc_W, fc_b)


# speedup vs baseline: 4.1282x; 1.2936x over previous
import functools

import jax
import jax.numpy as jnp
from jax import lax
from jax.experimental import pallas as pl
from jax.experimental.pallas import tpu as pltpu
from jax.experimental.pallas import tpu_sc as plsc

_N = 10000
_E = 320000
_DH = 256
_G = 64
_EPS = 1e-5
_NR = 10240
_TRASH = _N
_BM = 1000
_GRID = _N // _BM
_NB = 16


def _sc_scatter(table, src3, dst3, zeros128, n_chunks):
    mesh = plsc.VectorSubcoreMesh(core_axis_name="c", subcore_axis_name="s")

    def body(table_h, src_h, dst_h, zeros_h, out_h, src_v, dst_v, buf, acc, sem):
        c = lax.axis_index("c")
        s = lax.axis_index("s")
        pltpu.sync_copy(zeros_h, buf)
        for j in range(_NR // 16 // 128):
            pltpu.sync_copy(buf, acc.at[pl.ds(s * (_NR // 16) + j * 128, 128)])
        plsc.subcore_barrier()

        def step(i, carry):
            pltpu.async_copy(table_h.at[src_v.at[i]], buf, sem).wait()
            pltpu.sync_copy(buf, acc.at[dst_v.at[i]], add=True)
            return carry

        for b in range(n_chunks // _NB):
            pltpu.sync_copy(src_h.at[c, s, pl.ds(b * _NB, _NB)], src_v)
            pltpu.sync_copy(dst_h.at[c, s, pl.ds(b * _NB, _NB)], dst_v)
            lax.fori_loop(0, _NB, step, 0)
        plsc.subcore_barrier()
        base = s * (_NR // 16)
        for j in range(_NR // 16 // 128):
            r0 = base + j * 128
            pltpu.sync_copy(acc.at[pl.ds(r0, 128)], buf)
            pltpu.sync_copy(buf, out_h.at[c, pl.ds(r0, 128)])

    k = pl.kernel(
        body,
        out_type=jax.ShapeDtypeStruct((2, _NR, 128), jnp.float32),
        mesh=mesh,
        scratch_types=[
            pltpu.VMEM((_NB, 128), jnp.int32),
            pltpu.VMEM((_NB, 128), jnp.int32),
            pltpu.VMEM((128, 128), jnp.float32),
            pltpu.VMEM_SHARED((_NR, 128), jnp.float32),
            pltpu.SemaphoreType.DMA,
        ],
    )
    return k(table, src3, dst3, zeros128)


def _tc_layer(h, agg, W1, b1, W2, b2, first):
    din = 128 if first else _DH

    def body(h_ref, a_ref, W1_ref, b1_ref, W2_ref, b2_ref, z2_ref, st_ref):
        i = pl.program_id(0)
        if first:
            z = h_ref[...] + a_ref[0] + a_ref[1]
        else:
            z = jnp.concatenate([h_ref[0] + a_ref[0], h_ref[1] + a_ref[1]], axis=1)
        z1 = jnp.maximum(
            jnp.dot(z, W1_ref[...], preferred_element_type=jnp.float32) + b1_ref[...], 0.0)
        z2 = jnp.dot(z1, W2_ref[...], preferred_element_type=jnp.float32) + b2_ref[...]
        z2_ref[...] = z2
        sblk = jnp.concatenate(
            [jnp.sum(z2, axis=0, keepdims=True),
             jnp.sum(z2 * z2, axis=0, keepdims=True),
             jnp.zeros((6, _DH), jnp.float32)], axis=0)

        @pl.when(i == 0)
        def _():
            st_ref[...] = sblk

        @pl.when(i != 0)
        def _():
            st_ref[...] = st_ref[...] + sblk

    if first:
        h_spec = pl.BlockSpec((_BM, din), lambda i: (i, 0))
    else:
        h_spec = pl.BlockSpec((2, _BM, 128), lambda i: (0, i, 0))
    return pl.pallas_call(
        body,
        grid=(_GRID,),
        in_specs=[
            h_spec,
            pl.BlockSpec((2, _BM, 128), lambda i: (0, i, 0)),
            pl.BlockSpec((din, _DH), lambda i: (0, 0)),
            pl.BlockSpec((1, _DH), lambda i: (0, 0)),
            pl.BlockSpec((_DH, _DH), lambda i: (0, 0)),
            pl.BlockSpec((1, _DH), lambda i: (0, 0)),
        ],
        out_specs=[
            pl.BlockSpec((_BM, _DH), lambda i: (i, 0)),
            pl.BlockSpec((8, _DH), lambda i: (0, 0)),
        ],
        out_shape=[
            jax.ShapeDtypeStruct((_N, _DH), jnp.float32),
            jax.ShapeDtypeStruct((8, _DH), jnp.float32),
        ],
    )(h, agg, W1, b1.reshape(1, _DH), W2, b2.reshape(1, _DH))


def _tc_bn(z2, st, gamma, beta):
    def body(z_ref, st_ref, g_ref, be_ref, o_ref):
        mean = st_ref[0] * (1.0 / _N)
        var = st_ref[1] * (1.0 / _N) - mean * mean
        inv = g_ref[0] * lax.rsqrt(var + _EPS)
        sh = be_ref[0] - mean * inv
        y = jnp.maximum(z_ref[...] * inv + sh, 0.0)
        o_ref[0] = y[:, :128]
        o_ref[1] = y[:, 128:]

    return pl.pallas_call(
        body,
        grid=(_GRID,),
        in_specs=[
            pl.BlockSpec((_BM, _DH), lambda i: (i, 0)),
            pl.BlockSpec((8, _DH), lambda i: (0, 0)),
            pl.BlockSpec((1, _DH), lambda i: (0, 0)),
            pl.BlockSpec((1, _DH), lambda i: (0, 0)),
        ],
        out_specs=pl.BlockSpec((2, _BM, 128), lambda i: (0, i, 0)),
        out_shape=jax.ShapeDtypeStruct((2, _N, 128), jnp.float32),
    )(z2, st, gamma.reshape(1, _DH), beta.reshape(1, _DH))


def _tc_pool(h, batch3, fc_W, fc_b):
    def body(h_ref, b_ref, W_ref, bb_ref, o_ref, sums, cnts):
        i = pl.program_id(0)
        hcat = jnp.concatenate([h_ref[0], h_ref[1]], axis=1)
        bcol = b_ref[0, 0, :].reshape(_BM, 1)
        gid = lax.broadcasted_iota(jnp.int32, (_BM, _G), 1)
        P = (bcol == gid).astype(jnp.float32)
        ps = lax.dot_general(P, hcat, (((0,), (0,)), ((), ())),
                             preferred_element_type=jnp.float32)
        pc = lax.dot_general(P, jnp.ones((_BM, 128), jnp.float32),
                             (((0,), (0,)), ((), ())),
                             preferred_element_type=jnp.float32)

        @pl.when(i == 0)
        def _():
            sums[...] = ps
            cnts[...] = pc

        @pl.when(i != 0)
        def _():
            sums[...] = sums[...] + ps
            cnts[...] = cnts[...] + pc

        @pl.when(i == _GRID - 1)
        def _():
            cnt = jnp.maximum(cnts[:, 0:1], 1.0)
            pooled = sums[...] / cnt
            o_ref[...] = jnp.dot(pooled, W_ref[...],
                                 preferred_element_type=jnp.float32) + bb_ref[...]

    return pl.pallas_call(
        body,
        grid=(_GRID,),
        in_specs=[
            pl.BlockSpec((2, _BM, 128), lambda i: (0, i, 0)),
            pl.BlockSpec((1, 1, _BM), lambda i: (i, 0, 0)),
            pl.BlockSpec((_DH, 128), lambda i: (0, 0)),
            pl.BlockSpec((1, 128), lambda i: (0, 0)),
        ],
        out_specs=pl.BlockSpec((_G, 128), lambda i: (0, 0)),
        out_shape=jax.ShapeDtypeStruct((_G, 128), jnp.float32),
        scratch_shapes=[
            pltpu.VMEM((_G, _DH), jnp.float32),
            pltpu.VMEM((_G, 128), jnp.float32),
        ],
    )(h, batch3, fc_W, fc_b.reshape(1, 128))


def kernel(x, edge_index, batch,
           l0_W1, l0_b1, l0_W2, l0_b2, l0_gamma, l0_beta,
           l1_W1, l1_b1, l1_W2, l1_b2, l1_gamma, l1_beta,
           l2_W1, l2_b1, l2_W2, l2_b2, l2_gamma, l2_beta,
           fc_W, fc_b):
    src = edge_index[0]
    dst = edge_index[1]

    def strided(v, pad_val, n_chunks_total):
        m = n_chunks_total * 128
        vp = jnp.pad(v, (0, m - v.shape[0]), constant_values=pad_val)
        return vp.reshape(128, n_chunks_total).T

    srcA = jnp.stack([strided(src[:_E // 2], 0, 1280),
                      strided(src[_E // 2:], 0, 1280)]).reshape(2, 16, 80, 128)
    dstA = jnp.stack([strided(dst[:_E // 2], _TRASH, 1280),
                      strided(dst[_E // 2:], _TRASH, 1280)]).reshape(2, 16, 80, 128)
    sb = strided(src, 0, 2560).reshape(16, 160, 128)
    db = strided(dst, _TRASH, 2560).reshape(16, 160, 128)
    srcB = jnp.stack([sb, sb + _N])
    dstB = jnp.stack([db, db])
    zeros128 = jnp.zeros((128, 128), jnp.float32)

    agg0 = _sc_scatter(x, srcA, dstA, zeros128, 80)
    z2, st = _tc_layer(x, agg0, l0_W1, l0_b1, l0_W2, l0_b2, first=True)
    h = _tc_bn(z2, st, l0_gamma, l0_beta)

    agg1 = _sc_scatter(h.reshape(2 * _N, 128), srcB, dstB, zeros128, 160)
    z2, st = _tc_layer(h, agg1, l1_W1, l1_b1, l1_W2, l1_b2, first=False)
    h = _tc_bn(z2, st, l1_gamma, l1_beta)

    agg2 = _sc_scatter(h.reshape(2 * _N, 128), srcB, dstB, zeros128, 160)
    z2, st = _tc_layer(h, agg2, l2_W1, l2_b1, l2_W2, l2_b2, first=False)
    h = _tc_bn(z2, st, l2_gamma, l2_beta)

    return _tc_pool(h, batch.reshape(_GRID, 1, _BM), fc_W, fc_b)


# speedup vs baseline: 4.1311x; 1.0007x over previous
import functools

import jax
import jax.numpy as jnp
from jax import lax
from jax.experimental import pallas as pl
from jax.experimental.pallas import tpu as pltpu
from jax.experimental.pallas import tpu_sc as plsc

_N = 10000
_E = 320000
_DH = 256
_G = 64
_EPS = 1e-5
_NR = 10240
_TRASH = _N
_BM = 1000
_GRID = _N // _BM
_NB = 16


def _sc_scatter(table, src3, dst3, zeros128, n_chunks):
    mesh = plsc.VectorSubcoreMesh(core_axis_name="c", subcore_axis_name="s")

    def body(table_h, src_h, dst_h, zeros_h, out_h,
             src_v, dst_v, buf0, buf1, acc, sem0, sem1):
        c = lax.axis_index("c")
        s = lax.axis_index("s")
        pltpu.sync_copy(zeros_h, buf0)
        for j in range(_NR // 16 // 128):
            pltpu.sync_copy(buf0, acc.at[pl.ds(s * (_NR // 16) + j * 128, 128)])
        plsc.subcore_barrier()

        def pair(j, carry):
            i0 = 2 * j
            pltpu.make_async_copy(table_h.at[src_v.at[i0]], buf0, sem0).wait()
            pltpu.async_copy(table_h.at[src_v.at[i0 + 1]], buf1, sem1)
            pltpu.sync_copy(buf0, acc.at[dst_v.at[i0]], add=True)
            pltpu.make_async_copy(table_h.at[src_v.at[i0 + 1]], buf1, sem1).wait()

            @pl.when(j < _NB // 2 - 1)
            def _():
                pltpu.async_copy(table_h.at[src_v.at[i0 + 2]], buf0, sem0)

            pltpu.sync_copy(buf1, acc.at[dst_v.at[i0 + 1]], add=True)
            return carry

        for b in range(n_chunks // _NB):
            pltpu.sync_copy(src_h.at[c, s, pl.ds(b * _NB, _NB)], src_v)
            pltpu.sync_copy(dst_h.at[c, s, pl.ds(b * _NB, _NB)], dst_v)
            pltpu.async_copy(table_h.at[src_v.at[0]], buf0, sem0)
            lax.fori_loop(0, _NB // 2, pair, 0)
        plsc.subcore_barrier()
        base = s * (_NR // 16)
        for j in range(_NR // 16 // 128):
            r0 = base + j * 128
            pltpu.sync_copy(acc.at[pl.ds(r0, 128)], buf0)
            pltpu.sync_copy(buf0, out_h.at[c, pl.ds(r0, 128)])

    k = pl.kernel(
        body,
        out_type=jax.ShapeDtypeStruct((2, _NR, 128), jnp.float32),
        mesh=mesh,
        scratch_types=[
            pltpu.VMEM((_NB, 128), jnp.int32),
            pltpu.VMEM((_NB, 128), jnp.int32),
            pltpu.VMEM((128, 128), jnp.float32),
            pltpu.VMEM((128, 128), jnp.float32),
            pltpu.VMEM_SHARED((_NR, 128), jnp.float32),
            pltpu.SemaphoreType.DMA,
            pltpu.SemaphoreType.DMA,
        ],
    )
    return k(table, src3, dst3, zeros128)


def _tc_layer(h, agg, W1, b1, W2, b2, first):
    din = 128 if first else _DH

    def body(h_ref, a_ref, W1_ref, b1_ref, W2_ref, b2_ref, z2_ref, st_ref):
        i = pl.program_id(0)
        if first:
            z = h_ref[...] + a_ref[0] + a_ref[1]
        else:
            z = jnp.concatenate([h_ref[0] + a_ref[0], h_ref[1] + a_ref[1]], axis=1)
        z1 = jnp.maximum(
            jnp.dot(z, W1_ref[...], preferred_element_type=jnp.float32) + b1_ref[...], 0.0)
        z2 = jnp.dot(z1, W2_ref[...], preferred_element_type=jnp.float32) + b2_ref[...]
        z2_ref[...] = z2
        sblk = jnp.concatenate(
            [jnp.sum(z2, axis=0, keepdims=True),
             jnp.sum(z2 * z2, axis=0, keepdims=True),
             jnp.zeros((6, _DH), jnp.float32)], axis=0)

        @pl.when(i == 0)
        def _():
            st_ref[...] = sblk

        @pl.when(i != 0)
        def _():
            st_ref[...] = st_ref[...] + sblk

    if first:
        h_spec = pl.BlockSpec((_BM, din), lambda i: (i, 0))
    else:
        h_spec = pl.BlockSpec((2, _BM, 128), lambda i: (0, i, 0))
    return pl.pallas_call(
        body,
        grid=(_GRID,),
        in_specs=[
            h_spec,
            pl.BlockSpec((2, _BM, 128), lambda i: (0, i, 0)),
            pl.BlockSpec((din, _DH), lambda i: (0, 0)),
            pl.BlockSpec((1, _DH), lambda i: (0, 0)),
            pl.BlockSpec((_DH, _DH), lambda i: (0, 0)),
            pl.BlockSpec((1, _DH), lambda i: (0, 0)),
        ],
        out_specs=[
            pl.BlockSpec((_BM, _DH), lambda i: (i, 0)),
            pl.BlockSpec((8, _DH), lambda i: (0, 0)),
        ],
        out_shape=[
            jax.ShapeDtypeStruct((_N, _DH), jnp.float32),
            jax.ShapeDtypeStruct((8, _DH), jnp.float32),
        ],
    )(h, agg, W1, b1.reshape(1, _DH), W2, b2.reshape(1, _DH))


def _tc_bn(z2, st, gamma, beta):
    def body(z_ref, st_ref, g_ref, be_ref, o_ref):
        mean = st_ref[0] * (1.0 / _N)
        var = st_ref[1] * (1.0 / _N) - mean * mean
        inv = g_ref[0] * lax.rsqrt(var + _EPS)
        sh = be_ref[0] - mean * inv
        y = jnp.maximum(z_ref[...] * inv + sh, 0.0)
        o_ref[0] = y[:, :128]
        o_ref[1] = y[:, 128:]

    return pl.pallas_call(
        body,
        grid=(_GRID,),
        in_specs=[
            pl.BlockSpec((_BM, _DH), lambda i: (i, 0)),
            pl.BlockSpec((8, _DH), lambda i: (0, 0)),
            pl.BlockSpec((1, _DH), lambda i: (0, 0)),
            pl.BlockSpec((1, _DH), lambda i: (0, 0)),
        ],
        out_specs=pl.BlockSpec((2, _BM, 128), lambda i: (0, i, 0)),
        out_shape=jax.ShapeDtypeStruct((2, _N, 128), jnp.float32),
    )(z2, st, gamma.reshape(1, _DH), beta.reshape(1, _DH))


def _tc_pool(h, batch3, fc_W, fc_b):
    def body(h_ref, b_ref, W_ref, bb_ref, o_ref, sums, cnts):
        i = pl.program_id(0)
        hcat = jnp.concatenate([h_ref[0], h_ref[1]], axis=1)
        bcol = b_ref[0, 0, :].reshape(_BM, 1)
        gid = lax.broadcasted_iota(jnp.int32, (_BM, _G), 1)
        P = (bcol == gid).astype(jnp.float32)
        ps = lax.dot_general(P, hcat, (((0,), (0,)), ((), ())),
                             preferred_element_type=jnp.float32)
        pc = lax.dot_general(P, jnp.ones((_BM, 128), jnp.float32),
                             (((0,), (0,)), ((), ())),
                             preferred_element_type=jnp.float32)

        @pl.when(i == 0)
        def _():
            sums[...] = ps
            cnts[...] = pc

        @pl.when(i != 0)
        def _():
            sums[...] = sums[...] + ps
            cnts[...] = cnts[...] + pc

        @pl.when(i == _GRID - 1)
        def _():
            cnt = jnp.maximum(cnts[:, 0:1], 1.0)
            pooled = sums[...] / cnt
            o_ref[...] = jnp.dot(pooled, W_ref[...],
                                 preferred_element_type=jnp.float32) + bb_ref[...]

    return pl.pallas_call(
        body,
        grid=(_GRID,),
        in_specs=[
            pl.BlockSpec((2, _BM, 128), lambda i: (0, i, 0)),
            pl.BlockSpec((1, 1, _BM), lambda i: (i, 0, 0)),
            pl.BlockSpec((_DH, 128), lambda i: (0, 0)),
            pl.BlockSpec((1, 128), lambda i: (0, 0)),
        ],
        out_specs=pl.BlockSpec((_G, 128), lambda i: (0, 0)),
        out_shape=jax.ShapeDtypeStruct((_G, 128), jnp.float32),
        scratch_shapes=[
            pltpu.VMEM((_G, _DH), jnp.float32),
            pltpu.VMEM((_G, 128), jnp.float32),
        ],
    )(h, batch3, fc_W, fc_b.reshape(1, 128))


def kernel(x, edge_index, batch,
           l0_W1, l0_b1, l0_W2, l0_b2, l0_gamma, l0_beta,
           l1_W1, l1_b1, l1_W2, l1_b2, l1_gamma, l1_beta,
           l2_W1, l2_b1, l2_W2, l2_b2, l2_gamma, l2_beta,
           fc_W, fc_b):
    src = edge_index[0]
    dst = edge_index[1]

    def strided(v, pad_val, n_chunks_total):
        m = n_chunks_total * 128
        vp = jnp.pad(v, (0, m - v.shape[0]), constant_values=pad_val)
        return vp.reshape(128, n_chunks_total).T

    srcA = jnp.stack([strided(src[:_E // 2], 0, 1280),
                      strided(src[_E // 2:], 0, 1280)]).reshape(2, 16, 80, 128)
    dstA = jnp.stack([strided(dst[:_E // 2], _TRASH, 1280),
                      strided(dst[_E // 2:], _TRASH, 1280)]).reshape(2, 16, 80, 128)
    sb = strided(src, 0, 2560).reshape(16, 160, 128)
    db = strided(dst, _TRASH, 2560).reshape(16, 160, 128)
    srcB = jnp.stack([sb, sb + _N])
    dstB = jnp.stack([db, db])
    zeros128 = jnp.zeros((128, 128), jnp.float32)

    agg0 = _sc_scatter(x, srcA, dstA, zeros128, 80)
    z2, st = _tc_layer(x, agg0, l0_W1, l0_b1, l0_W2, l0_b2, first=True)
    h = _tc_bn(z2, st, l0_gamma, l0_beta)

    agg1 = _sc_scatter(h.reshape(2 * _N, 128), srcB, dstB, zeros128, 160)
    z2, st = _tc_layer(h, agg1, l1_W1, l1_b1, l1_W2, l1_b2, first=False)
    h = _tc_bn(z2, st, l1_gamma, l1_beta)

    agg2 = _sc_scatter(h.reshape(2 * _N, 128), srcB, dstB, zeros128, 160)
    z2, st = _tc_layer(h, agg2, l2_W1, l2_b1, l2_W2, l2_b2, first=False)
    h = _tc_bn(z2, st, l2_gamma, l2_beta)

    return _tc_pool(h, batch.reshape(_GRID, 1, _BM), fc_W, fc_b)
